# baseline (speedup 1.0000x reference)
import functools
import jax
import jax.numpy as jnp
from jax import lax
from jax.experimental import pallas as pl
from jax.experimental.pallas import tpu as pltpu
from jax.experimental.pallas import tpu_sc as plsc

_MARGIN = 1.0
_B, _C, _D = 1024, 1000, 64

_info = plsc.get_sparse_core_info()
_NC, _NS, _L = _info.num_cores, _info.num_subcores, _info.num_lanes
_NW = _NC * _NS
_BPW = _B // _NW


def _sc_body(lab_hbm, tabp_hbm, out_hbm, idx_v, idx2_v, rows_v, sem):
    wid = lax.axis_index("s") * _NC + lax.axis_index("c")
    base = wid * _BPW
    pltpu.sync_copy(lab_hbm.at[pl.ds(base, _BPW)], idx_v)
    for j in range(_BPW // _L):
        idx2_v[pl.ds(j * _L, _L)] = idx_v[pl.ds(j * _L, _L)] >> 1
    pltpu.async_copy(tabp_hbm.at[idx2_v], rows_v, sem).wait()
    pltpu.sync_copy(rows_v, out_hbm.at[pl.ds(base, _BPW)])


_sc_gather = functools.partial(
    pl.kernel,
    mesh=plsc.VectorSubcoreMesh(core_axis_name="c", subcore_axis_name="s"),
    out_type=jax.ShapeDtypeStruct((_B, 2 * _D), jnp.float32),
    scratch_types=[
        pltpu.VMEM((_BPW,), jnp.int32),
        pltpu.VMEM((_BPW,), jnp.int32),
        pltpu.VMEM((_BPW, 2 * _D), jnp.float32),
        pltpu.SemaphoreType.DMA,
    ],
)(_sc_body)


def _tc_kernel(woT_ref, lab_ref, tab_ref, embT_ref, out_ref):
    B = woT_ref.shape[1]
    C = tab_ref.shape[0]
    D = woT_ref.shape[0]
    woT = woT_ref[:]
    x2 = jnp.sum(woT * woT, axis=0, keepdims=True)
    inv = jax.lax.rsqrt(jnp.maximum(x2, 1e-24))
    wnT = woT * (-2.0 * inv)
    xn2 = x2 * (inv * inv)
    rhs = jnp.concatenate([wnT, jnp.ones((1, B), jnp.float32)], axis=0)

    tab = tab_ref[:]
    t2 = jnp.sum(tab * tab, axis=1, keepdims=True)
    lhs = jnp.concatenate([tab, t2], axis=1)
    d2p = jnp.dot(lhs, rhs, preferred_element_type=jnp.float32)

    lab = lab_ref[:]
    rows = jax.lax.broadcasted_iota(jnp.int32, (C, B), 0)
    is_lab = rows == lab
    min_d2 = jnp.min(jnp.where(is_lab, jnp.inf, d2p), axis=0, keepdims=True) + xn2

    embT = embT_ref[:]
    odd = (lab & 1) == 1
    emb_lo = embT[:D, :]
    emb_hi = embT[D:, :]
    dot_lo = jnp.sum(woT * emb_lo, axis=0, keepdims=True)
    dot_hi = jnp.sum(woT * emb_hi, axis=0, keepdims=True)
    t2_lo = jnp.sum(emb_lo * emb_lo, axis=0, keepdims=True)
    t2_hi = jnp.sum(emb_hi * emb_hi, axis=0, keepdims=True)
    dotwt = jnp.where(odd, dot_hi, dot_lo)
    t2l = jnp.where(odd, t2_hi, t2_lo)
    lab_d2 = xn2 + t2l + (-2.0 * inv) * dotwt

    lab_d = lab_d2 * jax.lax.rsqrt(jnp.maximum(lab_d2, 1e-30))
    min_d = min_d2 * jax.lax.rsqrt(jnp.maximum(min_d2, 1e-30))
    s = jnp.sum(lab_d - min_d, axis=1, keepdims=True)
    out_ref[:, :] = _MARGIN + s / B


def kernel(WO, label, table):
    B, _ = WO.shape
    lab32 = label.astype(jnp.int32)
    tabp = table.reshape(_C // 2, 2 * _D)
    emb = _sc_gather(lab32, tabp)
    out = pl.pallas_call(
        _tc_kernel,
        out_shape=jax.ShapeDtypeStruct((1, 1), jnp.float32),
    )(WO.T, lab32.reshape(1, B), table, emb.T)
    return out[0, 0]


# speedup vs baseline: 5.4147x; 5.4147x over previous
import jax
import jax.numpy as jnp
from jax.experimental import pallas as pl

_MARGIN = 1.0


def _loss_kernel(woT_ref, lab_ref, tab_ref, out_ref):
    B = woT_ref.shape[1]
    C = tab_ref.shape[0]
    woT = woT_ref[:]
    x2 = jnp.sum(woT * woT, axis=0, keepdims=True)
    inv = jax.lax.rsqrt(jnp.maximum(x2, 1e-24))
    wnT = woT * (-2.0 * inv)
    xn2 = x2 * (inv * inv)
    rhs = jnp.concatenate([wnT, jnp.ones((1, B), jnp.float32)], axis=0)

    tab = tab_ref[:]
    t2 = jnp.sum(tab * tab, axis=1, keepdims=True)
    lhs = jnp.concatenate([tab, t2], axis=1)
    d2p = jnp.dot(lhs, rhs, preferred_element_type=jnp.float32)

    lab = lab_ref[:]
    rows = jax.lax.broadcasted_iota(jnp.int32, (C, B), 0)
    is_lab = rows == lab
    lab_d2 = jnp.sum(jnp.where(is_lab, d2p, 0.0), axis=0, keepdims=True) + xn2
    min_d2 = jnp.min(jnp.where(is_lab, jnp.inf, d2p), axis=0, keepdims=True) + xn2
    lab_d = lab_d2 * jax.lax.rsqrt(jnp.maximum(lab_d2, 1e-30))
    min_d = min_d2 * jax.lax.rsqrt(jnp.maximum(min_d2, 1e-30))
    s = jnp.sum(lab_d - min_d, axis=1, keepdims=True)
    out_ref[:, :] = _MARGIN + s / B


def kernel(WO, label, table):
    B, _ = WO.shape
    out = pl.pallas_call(
        _loss_kernel,
        out_shape=jax.ShapeDtypeStruct((1, 1), jnp.float32),
    )(WO.T, label.astype(jnp.int32).reshape(1, B), table)
    return out[0, 0]
